# baseline (speedup 1.0000x reference)
import functools

import jax
import jax.numpy as jnp
from jax import lax
from jax.experimental import pallas as pl
from jax.experimental.pallas import tpu as pltpu
from jax.experimental.pallas import tpu_sc as plsc


PATCH_LEN = 32
NW = 32


def _sc_attr_body(w_hbm, dist2_hbm, ea_hbm, wv, dist_v, tmpl_v, sem):
    wid = lax.axis_index("s") * 2 + lax.axis_index("c")
    e2 = dist_v.shape[0]
    tw = tmpl_v.shape[0]
    pltpu.sync_copy(w_hbm, wv)
    pltpu.sync_copy(dist2_hbm, dist_v)
    for j in range(e2 // 16):
        idx = dist_v[pl.ds(j * 16, 16)]
        tmpl_v[pl.ds(j * 16, 16)] = plsc.load_gather(wv, [idx])
    sz = e2
    while sz < tw:
        for t in range(sz // 16):
            tmpl_v[pl.ds(sz + t * 16, 16)] = tmpl_v[pl.ds(t * 16, 16)]
        sz *= 2
    chunks = ea_hbm.shape[0] // (tw * NW)
    copies = [
        pltpu.make_async_copy(
            tmpl_v, ea_hbm.at[pl.ds((wid * chunks + j) * tw, tw)], sem)
        for j in range(chunks)
    ]
    for c in copies:
        c.start()
    for c in copies:
        c.wait()


def _tc_body(tY_ref, iq_ref, nf_ref, ei_ref, bt_ref):
    i = pl.program_id(0)

    reps = ei_ref.shape[0] // tY_ref.shape[0]
    rep = jax.lax.broadcasted_iota(jnp.int32, (reps, 1, 1), 0)
    ei = tY_ref[...][None] + (rep + i * reps) * (64 * PATCH_LEN)
    ei_ref[...] = ei.reshape(ei_ref.shape)

    bp = bt_ref.shape[0]
    p = jax.lax.broadcasted_iota(jnp.int32, (bp, 128), 0)
    c = jax.lax.broadcasted_iota(jnp.int32, (bp, 128), 1)
    bt_ref[...] = (i * bp + p) * 4 + jax.lax.shift_right_logical(c, 5)

    x = iq_ref[...]
    bb, _, L = x.shape
    z = x.reshape(bb, 2, L // 128, 128).transpose(0, 2, 1, 3)
    nf_ref[...] = z.reshape(bb * (L // 128) * 2, 128)


def kernel(iq_signal, edge_weights, edge_index, edge_distance):
    B, _, L = iq_signal.shape
    P = L // PATCH_LEN
    G = B * P
    E = edge_index.shape[1]
    idt = edge_index.dtype

    offs = (jnp.arange(64, dtype=jnp.int32) * PATCH_LEN)[None, :, None]
    v = (edge_index.astype(jnp.int32)[:, None, :] + offs).reshape(2, 64 * E)
    tY = v.reshape(2, 64 * E // 128, 128).transpose(1, 0, 2).reshape(E, 128)

    STEPS = 16
    YR = 2 * G * E // 128
    BR = G * PATCH_LEN // 128
    ZR = 2 * B * L // 128

    z, ei, bt = pl.pallas_call(
        _tc_body,
        grid=(STEPS,),
        in_specs=[
            pl.BlockSpec((E, 128), lambda i: (0, 0)),
            pl.BlockSpec((B // STEPS, 2, L), lambda i: (i, 0, 0)),
        ],
        out_specs=[
            pl.BlockSpec((ZR // STEPS, 128), lambda i: (i, 0)),
            pl.BlockSpec((YR // STEPS, 128), lambda i: (i, 0)),
            pl.BlockSpec((BR // STEPS, 128), lambda i: (i, 0)),
        ],
        out_shape=[
            jax.ShapeDtypeStruct((ZR, 128), jnp.float32),
            jax.ShapeDtypeStruct((YR, 128), jnp.int32),
            jax.ShapeDtypeStruct((BR, 128), jnp.int32),
        ],
    )(tY, iq_signal)

    w16 = jnp.pad(edge_weights, (0, 8))
    dist2 = jnp.tile(edge_distance.astype(jnp.int32), 2)
    ea = pl.kernel(
        _sc_attr_body,
        out_type=jax.ShapeDtypeStruct((G * E,), jnp.float32),
        mesh=plsc.VectorSubcoreMesh(core_axis_name="c", subcore_axis_name="s"),
        scratch_types=[
            pltpu.VMEM((16,), jnp.float32),
            pltpu.VMEM((2 * E,), jnp.int32),
            pltpu.VMEM((16 * E,), jnp.float32),
            pltpu.SemaphoreType.DMA,
        ],
        compiler_params=pltpu.CompilerParams(needs_layout_passes=False),
    )(w16, dist2)

    node_features = z.reshape(B * L // 128, 2, 128).transpose(0, 2, 1).reshape(
        B * L, 2)
    batch_edge_index = ei.reshape(G * E // 128, 2, 128).transpose(1, 0, 2).reshape(
        2, G * E).astype(idt)
    batch_edge_attr = ea
    batch = bt.reshape(G * PATCH_LEN).astype(idt)
    return (node_features, batch_edge_index, batch_edge_attr, batch)


# speedup vs baseline: 1.0337x; 1.0337x over previous
import functools

import jax
import jax.numpy as jnp
from jax import lax
from jax.experimental import pallas as pl
from jax.experimental.pallas import tpu as pltpu
from jax.experimental.pallas import tpu_sc as plsc


PATCH_LEN = 32
NW = 32


def _sc_attr_body(w_hbm, dist2_hbm, ea_hbm, wv, dist_v, tmpl_v, sem):
    wid = lax.axis_index("s") * 2 + lax.axis_index("c")
    e2 = dist_v.shape[0]
    tw = tmpl_v.shape[0]
    pltpu.sync_copy(w_hbm, wv)
    pltpu.sync_copy(dist2_hbm, dist_v)
    for j in range(e2 // 16):
        idx = dist_v[pl.ds(j * 16, 16)]
        tmpl_v[pl.ds(j * 16, 16)] = plsc.load_gather(wv, [idx])
    sz = e2
    while sz < tw:
        iters = sz // 16
        if iters <= 64:
            for t in range(iters):
                tmpl_v[pl.ds(sz + t * 16, 16)] = tmpl_v[pl.ds(t * 16, 16)]
        else:
            u = 8 if iters % 8 == 0 else (4 if iters % 4 == 0 else 2)
            def cp(t, c, base=sz, uu=u):
                for k in range(uu):
                    off = (t * uu + k) * 16
                    tmpl_v[pl.ds(base + off, 16)] = tmpl_v[pl.ds(off, 16)]
                return c
            lax.fori_loop(0, iters // u, cp, 0)
        sz *= 2
    chunks = ea_hbm.shape[0] // (tw * NW)
    copies = [
        pltpu.make_async_copy(
            tmpl_v, ea_hbm.at[pl.ds((wid * chunks + j) * tw, tw)], sem)
        for j in range(chunks)
    ]
    for c in copies:
        c.start()
    for c in copies:
        c.wait()


def _tc_body(tY_ref, iq_ref, nf_ref, ei_ref, bt_ref):
    i = pl.program_id(0)

    reps = ei_ref.shape[0] // tY_ref.shape[0]
    rep = jax.lax.broadcasted_iota(jnp.int32, (reps, 1, 1), 0)
    ei = tY_ref[...][None] + (rep + i * reps) * (64 * PATCH_LEN)
    ei_ref[...] = ei.reshape(ei_ref.shape)

    bp = bt_ref.shape[0]
    p = jax.lax.broadcasted_iota(jnp.int32, (bp, 128), 0)
    c = jax.lax.broadcasted_iota(jnp.int32, (bp, 128), 1)
    bt_ref[...] = (i * bp + p) * 4 + jax.lax.shift_right_logical(c, 5)

    x = iq_ref[...]
    bb, _, L = x.shape
    z = x.reshape(bb, 2, L // 128, 128).transpose(0, 2, 1, 3)
    nf_ref[...] = z.reshape(bb * (L // 128) * 2, 128)


def kernel(iq_signal, edge_weights, edge_index, edge_distance):
    B, _, L = iq_signal.shape
    P = L // PATCH_LEN
    G = B * P
    E = edge_index.shape[1]
    idt = edge_index.dtype

    offs = (jnp.arange(64, dtype=jnp.int32) * PATCH_LEN)[None, :, None]
    v = (edge_index.astype(jnp.int32)[:, None, :] + offs).reshape(2, 64 * E)
    tY = v.reshape(2, 64 * E // 128, 128).transpose(1, 0, 2).reshape(E, 128)

    STEPS = 16
    YR = 2 * G * E // 128
    BR = G * PATCH_LEN // 128
    ZR = 2 * B * L // 128

    z, ei, bt = pl.pallas_call(
        _tc_body,
        grid=(STEPS,),
        in_specs=[
            pl.BlockSpec((E, 128), lambda i: (0, 0)),
            pl.BlockSpec((B // STEPS, 2, L), lambda i: (i, 0, 0)),
        ],
        out_specs=[
            pl.BlockSpec((ZR // STEPS, 128), lambda i: (i, 0)),
            pl.BlockSpec((YR // STEPS, 128), lambda i: (i, 0)),
            pl.BlockSpec((BR // STEPS, 128), lambda i: (i, 0)),
        ],
        out_shape=[
            jax.ShapeDtypeStruct((ZR, 128), jnp.float32),
            jax.ShapeDtypeStruct((YR, 128), jnp.int32),
            jax.ShapeDtypeStruct((BR, 128), jnp.int32),
        ],
    )(tY, iq_signal)

    w16 = jnp.pad(edge_weights, (0, 8))
    dist2 = jnp.tile(edge_distance.astype(jnp.int32), 2)
    ea = pl.kernel(
        _sc_attr_body,
        out_type=jax.ShapeDtypeStruct((G * E,), jnp.float32),
        mesh=plsc.VectorSubcoreMesh(core_axis_name="c", subcore_axis_name="s"),
        scratch_types=[
            pltpu.VMEM((16,), jnp.float32),
            pltpu.VMEM((2 * E,), jnp.int32),
            pltpu.VMEM((64 * E,), jnp.float32),
            pltpu.SemaphoreType.DMA,
        ],
        compiler_params=pltpu.CompilerParams(needs_layout_passes=False),
    )(w16, dist2)

    node_features = z.reshape(B * L // 128, 2, 128).transpose(0, 2, 1).reshape(
        B * L, 2)
    batch_edge_index = ei.reshape(G * E // 128, 2, 128).transpose(1, 0, 2).reshape(
        2, G * E).astype(idt)
    batch_edge_attr = ea
    batch = bt.reshape(G * PATCH_LEN).astype(idt)
    return (node_features, batch_edge_index, batch_edge_attr, batch)


# speedup vs baseline: 1.4925x; 1.4438x over previous
import jax
import jax.numpy as jnp
from jax.experimental import pallas as pl


PATCH_LEN = 32


def _tc_body(tY_ref, dist_ref, w_ref, iq_ref, nf_ref, ei_ref, ea_ref, bt_ref):
    i = pl.program_id(0)

    reps = ei_ref.shape[0] // tY_ref.shape[0]
    rep = jax.lax.broadcasted_iota(jnp.int32, (reps, 1, 1), 0)
    ei = tY_ref[...][None] + (rep + i * reps) * (64 * PATCH_LEN)
    ei_ref[...] = ei.reshape(ei_ref.shape)

    d = dist_ref[...]
    attr = jnp.zeros(d.shape, jnp.float32)
    for k in range(8):
        attr = attr + jnp.where(d == k, w_ref[0, k], 0.0)
    areps = ea_ref.shape[0] // d.shape[0]
    ea_ref[...] = jnp.broadcast_to(attr[None], (areps,) + d.shape).reshape(
        ea_ref.shape)

    bp = bt_ref.shape[0]
    p = jax.lax.broadcasted_iota(jnp.int32, (bp, 128), 0)
    c = jax.lax.broadcasted_iota(jnp.int32, (bp, 128), 1)
    bt_ref[...] = (i * bp + p) * 4 + jax.lax.shift_right_logical(c, 5)

    x = iq_ref[...]
    bb, _, L = x.shape
    z = x.reshape(bb, 2, L // 128, 128).transpose(0, 2, 1, 3)
    nf_ref[...] = z.reshape(bb * (L // 128) * 2, 128)


def kernel(iq_signal, edge_weights, edge_index, edge_distance):
    B, _, L = iq_signal.shape
    P = L // PATCH_LEN
    G = B * P
    E = edge_index.shape[1]
    idt = edge_index.dtype

    offs = (jnp.arange(64, dtype=jnp.int32) * PATCH_LEN)[None, :, None]
    v = (edge_index.astype(jnp.int32)[:, None, :] + offs).reshape(2, 64 * E)
    tY = v.reshape(2, 64 * E // 128, 128).transpose(1, 0, 2).reshape(E, 128)
    dist = jnp.tile(edge_distance.astype(jnp.int32), 128).reshape(E, 128)
    w = jnp.pad(edge_weights, (0, 120)).reshape(1, 128)

    STEPS = 16
    YR = 2 * G * E // 128
    AR = G * E // 128
    BR = G * PATCH_LEN // 128
    ZR = 2 * B * L // 128

    z, ei, ea, bt = pl.pallas_call(
        _tc_body,
        grid=(STEPS,),
        in_specs=[
            pl.BlockSpec((E, 128), lambda i: (0, 0)),
            pl.BlockSpec((E, 128), lambda i: (0, 0)),
            pl.BlockSpec((1, 128), lambda i: (0, 0)),
            pl.BlockSpec((B // STEPS, 2, L), lambda i: (i, 0, 0)),
        ],
        out_specs=[
            pl.BlockSpec((ZR // STEPS, 128), lambda i: (i, 0)),
            pl.BlockSpec((YR // STEPS, 128), lambda i: (i, 0)),
            pl.BlockSpec((AR // STEPS, 128), lambda i: (i, 0)),
            pl.BlockSpec((BR // STEPS, 128), lambda i: (i, 0)),
        ],
        out_shape=[
            jax.ShapeDtypeStruct((ZR, 128), jnp.float32),
            jax.ShapeDtypeStruct((YR, 128), jnp.int32),
            jax.ShapeDtypeStruct((AR, 128), jnp.float32),
            jax.ShapeDtypeStruct((BR, 128), jnp.int32),
        ],
    )(tY, dist, w, iq_signal)

    node_features = z.reshape(B * L // 128, 2, 128).transpose(0, 2, 1).reshape(
        B * L, 2)
    batch_edge_index = ei.reshape(G * E // 128, 2, 128).transpose(1, 0, 2).reshape(
        2, G * E).astype(idt)
    batch_edge_attr = ea.reshape(G * E)
    batch = bt.reshape(G * PATCH_LEN).astype(idt)
    return (node_features, batch_edge_index, batch_edge_attr, batch)
